# baseline (speedup 1.0000x reference)
import functools
import math

import jax
import jax.numpy as jnp
from jax import lax
from jax.experimental import pallas as pl
from jax.experimental.pallas import tpu as pltpu
from jax.experimental.pallas import tpu_sc as plsc

_N = 262144
_M = 2 * _N
_B = 128
_L = 16
_NC = 2
_NS = 16
_NW = _NC * _NS
_C = 1024
_NBUF = 2
_NSPLIT_CALLS = 2
_NP = _N // _NSPLIT_CALLS
_LOG_SPLIT = math.log(0.8 * 2)
_GRAD_THRESHOLD = 0.5
_MAX_THRESHOLD = 0.1 * 5.0


def _densify_body(xyz_h, scal_h, rot_h, grads_h, n0_h, n1_h,
                  ox0_h, ox1_h, os0_h, os1_h, or0_h, or1_h, *scratch):
  rw = _NP // _NW
  nchunk = rw // _C
  in_bufs = [scratch[6 * b:6 * b + 6] for b in range(_NBUF)]
  out_bufs = [scratch[12 + 4 * b:12 + 4 * b + 4] for b in range(_NBUF)]
  in_sems = scratch[20:22]
  out_sems = scratch[22:24]

  cid = lax.axis_index("c")
  sid = lax.axis_index("s")
  wid = sid * _NC + cid
  base = wid * rw

  def start_in(k, b):
    off = base + k * _C
    xyz_v, scal_v, rot_v, grads_v, n0_v, n1_v = in_bufs[b]
    sem = in_sems[b]
    h = []
    for c in range(3):
      h.append(pltpu.async_copy(xyz_h.at[pl.ds(c * _NP + off, _C)],
                                xyz_v.at[pl.ds(c * _C, _C)], sem))
      h.append(pltpu.async_copy(scal_h.at[pl.ds(c * _NP + off, _C)],
                                scal_v.at[pl.ds(c * _C, _C)], sem))
      h.append(pltpu.async_copy(n0_h.at[pl.ds(c * _NP + off, _C)],
                                n0_v.at[pl.ds(c * _C, _C)], sem))
      h.append(pltpu.async_copy(n1_h.at[pl.ds(c * _NP + off, _C)],
                                n1_v.at[pl.ds(c * _C, _C)], sem))
    h.append(pltpu.async_copy(rot_h.at[pl.ds(4 * off, 4 * _C)], rot_v, sem))
    h.append(pltpu.async_copy(grads_h.at[pl.ds(off, _C)], grads_v, sem))
    return h

  def start_out(k, b):
    off = base + k * _C
    ox0_v, ox1_v, os_v, or_v = out_bufs[b]
    sem = out_sems[b]
    return [
        pltpu.async_copy(ox0_v, ox0_h.at[pl.ds(4 * off, 4 * _C)], sem),
        pltpu.async_copy(ox1_v, ox1_h.at[pl.ds(4 * off, 4 * _C)], sem),
        pltpu.async_copy(os_v, os0_h.at[pl.ds(4 * off, 4 * _C)], sem),
        pltpu.async_copy(os_v, os1_h.at[pl.ds(4 * off, 4 * _C)], sem),
        pltpu.async_copy(or_v, or0_h.at[pl.ds(4 * off, 4 * _C)], sem),
        pltpu.async_copy(or_v, or1_h.at[pl.ds(4 * off, 4 * _C)], sem),
    ]

  def compute(b):
    xyz_v, scal_v, rot_v, grads_v, n0_v, n1_v = in_bufs[b]
    ox0_v, ox1_v, os_v, or_v = out_bufs[b]

    def step(i, scarry):
      j = i * _L
      ji = (i // (_B // _L)) * (4 * _B) + (i % (_B // _L)) * _L

      s0 = scal_v[pl.ds(j, _L)]
      s1 = scal_v[pl.ds(_C + j, _L)]
      s2 = scal_v[pl.ds(2 * _C + j, _L)]
      e0 = jnp.exp(s0)
      e1 = jnp.exp(s1)
      e2 = jnp.exp(s2)
      g = grads_v[pl.ds(j, _L)]
      smax = jnp.maximum(jnp.maximum(e0, e1), e2)
      sel = (g >= _GRAD_THRESHOLD) & (smax > _MAX_THRESHOLD)
      m = jnp.where(sel, jnp.float32(1.0), jnp.float32(0.0))

      q0 = rot_v[pl.ds(ji, _L)]
      q1 = rot_v[pl.ds(ji + _B, _L)]
      q2 = rot_v[pl.ds(ji + 2 * _B, _L)]
      q3 = rot_v[pl.ds(ji + 3 * _B, _L)]
      q11 = q1 * q1
      q22 = q2 * q2
      q33 = q3 * q3
      ss = q0 * q0 + q11 + q22 + q33
      t = jnp.float32(2.0) / ss
      q12 = q1 * q2
      q13 = q1 * q3
      q23 = q2 * q3
      q01 = q0 * q1
      q02 = q0 * q2
      q03 = q0 * q3
      r00 = jnp.float32(1.0) - t * (q22 + q33)
      r01 = t * (q12 - q03)
      r02 = t * (q13 + q02)
      r10 = t * (q12 + q03)
      r11 = jnp.float32(1.0) - t * (q11 + q33)
      r12 = t * (q23 - q01)
      r20 = t * (q13 - q02)
      r21 = t * (q23 + q01)
      r22 = jnp.float32(1.0) - t * (q11 + q22)

      px = xyz_v[pl.ds(j, _L)]
      py = xyz_v[pl.ds(_C + j, _L)]
      pz = xyz_v[pl.ds(2 * _C + j, _L)]

      for n_v, ox_v in ((n0_v, ox0_v), (n1_v, ox1_v)):
        a0 = n_v[pl.ds(j, _L)] * e0
        a1 = n_v[pl.ds(_C + j, _L)] * e1
        a2 = n_v[pl.ds(2 * _C + j, _L)] * e2
        ox_v[pl.ds(ji, _L)] = (r00 * a0 + r01 * a1 + r02 * a2 + px) * m
        ox_v[pl.ds(ji + _B, _L)] = (r10 * a0 + r11 * a1 + r12 * a2 + py) * m
        ox_v[pl.ds(ji + 2 * _B, _L)] = (r20 * a0 + r21 * a1 + r22 * a2
                                        + pz) * m

      os_v[pl.ds(ji, _L)] = (s0 - _LOG_SPLIT) * m
      os_v[pl.ds(ji + _B, _L)] = (s1 - _LOG_SPLIT) * m
      os_v[pl.ds(ji + 2 * _B, _L)] = (s2 - _LOG_SPLIT) * m
      or_v[pl.ds(ji, _L)] = q0 * m
      or_v[pl.ds(ji + _B, _L)] = q1 * m
      or_v[pl.ds(ji + 2 * _B, _L)] = q2 * m
      or_v[pl.ds(ji + 3 * _B, _L)] = q3 * m
      return scarry

    lax.fori_loop(0, _C // _L, step, 0)

  pending_in = {0: start_in(0, 0)}
  pending_out = {}
  for k in range(nchunk):
    b = k % _NBUF
    if k + 1 < nchunk:
      pending_in[k + 1] = start_in(k + 1, (k + 1) % _NBUF)
    for h in pending_in.pop(k):
      h.wait()
    if k - _NBUF in pending_out:
      for h in pending_out.pop(k - _NBUF):
        h.wait()
    compute(b)
    pending_out[k] = start_out(k, b)
  for k in sorted(pending_out):
    for h in pending_out[k]:
      h.wait()


_densify = functools.partial(
    pl.kernel,
    out_type=tuple(
        jax.ShapeDtypeStruct((4 * _NP,), jnp.float32) for _ in range(6)),
    mesh=plsc.VectorSubcoreMesh(core_axis_name="c", subcore_axis_name="s"),
    compiler_params=pltpu.CompilerParams(
        needs_layout_passes=False, use_tc_tiling_on_sc=False),
    scratch_types=(
        [t for _ in range(_NBUF) for t in (
            pltpu.VMEM((3 * _C,), jnp.float32),
            pltpu.VMEM((3 * _C,), jnp.float32),
            pltpu.VMEM((4 * _C,), jnp.float32),
            pltpu.VMEM((_C,), jnp.float32),
            pltpu.VMEM((3 * _C,), jnp.float32),
            pltpu.VMEM((3 * _C,), jnp.float32),
        )] +
        [t for _ in range(_NBUF) for t in (
            pltpu.VMEM((4 * _C,), jnp.float32),
            pltpu.VMEM((4 * _C,), jnp.float32),
            pltpu.VMEM((4 * _C,), jnp.float32),
            pltpu.VMEM((4 * _C,), jnp.float32),
        )] +
        [pltpu.SemaphoreType.DMA] * 4
    ),
)(_densify_body)


def kernel(xyz, scaling, rotation, grads, noise):
  xyz_t = xyz.T
  scal_t = scaling.T
  noise_t = noise.T
  grads_f = grads.reshape(-1)
  rot_blk = rotation.reshape(_N // _B, _B, 4)
  nb = _NP // _B
  parts = []
  for p in range(_NSPLIT_CALLS):
    lo = p * _NP
    hi = lo + _NP
    parts.append(_densify(
        xyz_t[:, lo:hi].reshape(-1),
        scal_t[:, lo:hi].reshape(-1),
        rot_blk[p * nb:(p + 1) * nb].transpose(0, 2, 1).reshape(-1),
        grads_f[lo:hi],
        noise_t[:, lo:hi].reshape(-1),
        noise_t[:, _N + lo:_N + hi].reshape(-1),
    ))

  def assemble(idx):
    pieces = [parts[p][idx + h]
              for h in range(2) for p in range(_NSPLIT_CALLS)]
    return jnp.concatenate(pieces)

  fxyz = assemble(0)
  fscal = assemble(2)
  frot = assemble(4)
  nxyz = fxyz.reshape(_M // _B, 4, _B)[:, :3, :].transpose(0, 2, 1)
  nscal = fscal.reshape(_M // _B, 4, _B)[:, :3, :].transpose(0, 2, 1)
  nrot = frot.reshape(_M // _B, 4, _B).transpose(0, 2, 1)
  return (nxyz.reshape(_M, 3), nscal.reshape(_M, 3), nrot.reshape(_M, 4))


# speedup vs baseline: 1.6062x; 1.6062x over previous
import functools
import math

import jax
import jax.numpy as jnp
from jax import lax
from jax.experimental import pallas as pl
from jax.experimental.pallas import tpu as pltpu
from jax.experimental.pallas import tpu_sc as plsc

_N = 262144
_M = 2 * _N
_B = 128
_L = 16
_NC = 2
_NS = 16
_NW = _NC * _NS
_RW = _N // _NW
_C = 1024
_NCHUNK = _RW // _C
_NBUF = 3
_LOG_SPLIT = math.log(0.8 * 2)
_GRAD_THRESHOLD = 0.5
_MAX_THRESHOLD = 0.1 * 5.0


def _densify_body(xyz_h, scal_h, rot_h, grads_h, noise_h,
                  oxyz_h, oscal_h, orot_h, *scratch):
  in_bufs = [scratch[6 * b:6 * b + 6] for b in range(_NBUF)]
  out_bufs = [scratch[6 * _NBUF + 4 * b:6 * _NBUF + 4 * b + 4] for b in range(_NBUF)]
  in_sems = scratch[6 * _NBUF + 4 * _NBUF:6 * _NBUF + 4 * _NBUF + _NBUF]
  out_sems = scratch[6 * _NBUF + 5 * _NBUF:6 * _NBUF + 6 * _NBUF]

  cid = lax.axis_index("c")
  sid = lax.axis_index("s")
  wid = sid * _NC + cid
  base = wid * _RW

  def start_in(k, b):
    off = base + k * _C
    xyz_v, scal_v, rot_v, grads_v, n0_v, n1_v = in_bufs[b]
    sem = in_sems[b]
    h = []
    for c in range(3):
      h.append(pltpu.async_copy(xyz_h.at[pl.ds(c * _N + off, _C)],
                                xyz_v.at[pl.ds(c * _C, _C)], sem))
      h.append(pltpu.async_copy(scal_h.at[pl.ds(c * _N + off, _C)],
                                scal_v.at[pl.ds(c * _C, _C)], sem))
      h.append(pltpu.async_copy(noise_h.at[pl.ds(c * _M + off, _C)],
                                n0_v.at[pl.ds(c * _C, _C)], sem))
      h.append(pltpu.async_copy(noise_h.at[pl.ds(c * _M + _N + off, _C)],
                                n1_v.at[pl.ds(c * _C, _C)], sem))
    h.append(pltpu.async_copy(rot_h.at[pl.ds(4 * off, 4 * _C)], rot_v, sem))
    h.append(pltpu.async_copy(grads_h.at[pl.ds(off, _C)], grads_v, sem))
    return h

  def start_out(k, b):
    off = base + k * _C
    ox0_v, ox1_v, os_v, or_v = out_bufs[b]
    sem = out_sems[b]
    return [
        pltpu.async_copy(ox0_v, oxyz_h.at[pl.ds(4 * off, 4 * _C)], sem),
        pltpu.async_copy(ox1_v, oxyz_h.at[pl.ds(4 * (_N + off), 4 * _C)],
                         sem),
        pltpu.async_copy(os_v, oscal_h.at[pl.ds(4 * off, 4 * _C)], sem),
        pltpu.async_copy(os_v, oscal_h.at[pl.ds(4 * (_N + off), 4 * _C)],
                         sem),
        pltpu.async_copy(or_v, orot_h.at[pl.ds(4 * off, 4 * _C)], sem),
        pltpu.async_copy(or_v, orot_h.at[pl.ds(4 * (_N + off), 4 * _C)],
                         sem),
    ]

  def compute(b):
    xyz_v, scal_v, rot_v, grads_v, n0_v, n1_v = in_bufs[b]
    ox0_v, ox1_v, os_v, or_v = out_bufs[b]

    def step(i, scarry):
      j = i * _L
      ji = (i // (_B // _L)) * (4 * _B) + (i % (_B // _L)) * _L

      s0 = scal_v[pl.ds(j, _L)]
      s1 = scal_v[pl.ds(_C + j, _L)]
      s2 = scal_v[pl.ds(2 * _C + j, _L)]
      e0 = jnp.exp(s0)
      e1 = jnp.exp(s1)
      e2 = jnp.exp(s2)
      g = grads_v[pl.ds(j, _L)]
      smax = jnp.maximum(jnp.maximum(e0, e1), e2)
      sel = (g >= _GRAD_THRESHOLD) & (smax > _MAX_THRESHOLD)
      m = jnp.where(sel, jnp.float32(1.0), jnp.float32(0.0))

      q0 = rot_v[pl.ds(ji, _L)]
      q1 = rot_v[pl.ds(ji + _B, _L)]
      q2 = rot_v[pl.ds(ji + 2 * _B, _L)]
      q3 = rot_v[pl.ds(ji + 3 * _B, _L)]
      q11 = q1 * q1
      q22 = q2 * q2
      q33 = q3 * q3
      ss = q0 * q0 + q11 + q22 + q33
      t = jnp.float32(2.0) / ss
      q12 = q1 * q2
      q13 = q1 * q3
      q23 = q2 * q3
      q01 = q0 * q1
      q02 = q0 * q2
      q03 = q0 * q3
      r00 = jnp.float32(1.0) - t * (q22 + q33)
      r01 = t * (q12 - q03)
      r02 = t * (q13 + q02)
      r10 = t * (q12 + q03)
      r11 = jnp.float32(1.0) - t * (q11 + q33)
      r12 = t * (q23 - q01)
      r20 = t * (q13 - q02)
      r21 = t * (q23 + q01)
      r22 = jnp.float32(1.0) - t * (q11 + q22)

      px = xyz_v[pl.ds(j, _L)]
      py = xyz_v[pl.ds(_C + j, _L)]
      pz = xyz_v[pl.ds(2 * _C + j, _L)]

      for n_v, ox_v in ((n0_v, ox0_v), (n1_v, ox1_v)):
        a0 = n_v[pl.ds(j, _L)] * e0
        a1 = n_v[pl.ds(_C + j, _L)] * e1
        a2 = n_v[pl.ds(2 * _C + j, _L)] * e2
        ox_v[pl.ds(ji, _L)] = (r00 * a0 + r01 * a1 + r02 * a2 + px) * m
        ox_v[pl.ds(ji + _B, _L)] = (r10 * a0 + r11 * a1 + r12 * a2 + py) * m
        ox_v[pl.ds(ji + 2 * _B, _L)] = (r20 * a0 + r21 * a1 + r22 * a2
                                        + pz) * m

      os_v[pl.ds(ji, _L)] = (s0 - _LOG_SPLIT) * m
      os_v[pl.ds(ji + _B, _L)] = (s1 - _LOG_SPLIT) * m
      os_v[pl.ds(ji + 2 * _B, _L)] = (s2 - _LOG_SPLIT) * m
      or_v[pl.ds(ji, _L)] = q0 * m
      or_v[pl.ds(ji + _B, _L)] = q1 * m
      or_v[pl.ds(ji + 2 * _B, _L)] = q2 * m
      or_v[pl.ds(ji + 3 * _B, _L)] = q3 * m
      return scarry

    lax.fori_loop(0, _C // _L, step, 0)

  pending_in = {k: start_in(k, k % _NBUF)
                for k in range(min(_NBUF - 1, _NCHUNK))}
  pending_out = {}
  for k in range(_NCHUNK):
    b = k % _NBUF
    if k + _NBUF - 1 < _NCHUNK:
      pending_in[k + _NBUF - 1] = start_in(k + _NBUF - 1,
                                           (k + _NBUF - 1) % _NBUF)
    for h in pending_in.pop(k):
      h.wait()
    if k - _NBUF in pending_out:
      for h in pending_out.pop(k - _NBUF):
        h.wait()
    compute(b)
    pending_out[k] = start_out(k, b)
  for k in sorted(pending_out):
    for h in pending_out[k]:
      h.wait()


_densify = functools.partial(
    pl.kernel,
    out_type=(
        jax.ShapeDtypeStruct((4 * _M,), jnp.float32),
        jax.ShapeDtypeStruct((4 * _M,), jnp.float32),
        jax.ShapeDtypeStruct((4 * _M,), jnp.float32),
    ),
    mesh=plsc.VectorSubcoreMesh(core_axis_name="c", subcore_axis_name="s"),
    compiler_params=pltpu.CompilerParams(
        needs_layout_passes=False, use_tc_tiling_on_sc=False),
    scratch_types=(
        [t for _ in range(_NBUF) for t in (
            pltpu.VMEM((3 * _C,), jnp.float32),
            pltpu.VMEM((3 * _C,), jnp.float32),
            pltpu.VMEM((4 * _C,), jnp.float32),
            pltpu.VMEM((_C,), jnp.float32),
            pltpu.VMEM((3 * _C,), jnp.float32),
            pltpu.VMEM((3 * _C,), jnp.float32),
        )] +
        [t for _ in range(_NBUF) for t in (
            pltpu.VMEM((4 * _C,), jnp.float32),
            pltpu.VMEM((4 * _C,), jnp.float32),
            pltpu.VMEM((4 * _C,), jnp.float32),
            pltpu.VMEM((4 * _C,), jnp.float32),
        )] +
        [pltpu.SemaphoreType.DMA] * (2 * _NBUF)
    ),
)(_densify_body)


def kernel(xyz, scaling, rotation, grads, noise):
  rot_f = rotation.reshape(_N // _B, _B, 4).transpose(0, 2, 1).reshape(-1)
  fxyz, fscal, frot = _densify(
      xyz.T.reshape(-1), scaling.T.reshape(-1), rot_f,
      grads.reshape(-1), noise.T.reshape(-1))
  nxyz = fxyz.reshape(_M // _B, 4, _B)[:, :3, :].transpose(0, 2, 1)
  nscal = fscal.reshape(_M // _B, 4, _B)[:, :3, :].transpose(0, 2, 1)
  nrot = frot.reshape(_M // _B, 4, _B).transpose(0, 2, 1)
  return (nxyz.reshape(_M, 3), nscal.reshape(_M, 3), nrot.reshape(_M, 4))
